# Initial kernel scaffold; baseline (speedup 1.0000x reference)
import functools

import jax
import jax.numpy as jnp
from jax import lax
from jax.experimental import pallas as pl
from jax.experimental.pallas import tpu as pltpu
from jax.experimental.pallas import tpu_sc as plsc

VOCAB = 100000
CHAR_DIM = 64
BATCH = 4096
SEQ_LEN = 200

_N = BATCH * SEQ_LEN
_LANE = 128
_NROWS = _N // _LANE
_NW = 32
_IROWS_W = _NROWS // _NW
_G = 8
_NBLK = _IROWS_W // _G


@functools.partial(
    pl.kernel,
    out_type=jax.ShapeDtypeStruct((_NROWS, _LANE, CHAR_DIM), jnp.float32),
    mesh=plsc.VectorSubcoreMesh(core_axis_name="c", subcore_axis_name="s"),
    scratch_types=[
        pltpu.VMEM((_G, _LANE), jnp.int32),
        pltpu.VMEM((_G, _LANE, CHAR_DIM), jnp.float32),
        pltpu.SemaphoreType.DMA,
    ],
)
def _emb_gather(idx_hbm, tab_hbm, out_hbm, idx_v, rows_v, sem):
    num_cores = 2
    wid = lax.axis_index("s") * num_cores + lax.axis_index("c")
    base = wid * _IROWS_W

    @pl.loop(0, _NBLK)
    def _blk(b):
        r0 = base + b * _G
        pltpu.sync_copy(idx_hbm.at[pl.ds(r0, _G)], idx_v)
        copies = [
            pltpu.async_copy(tab_hbm.at[idx_v.at[j]], rows_v.at[j], sem)
            for j in range(_G)
        ]
        for c in copies:
            c.wait()
        pltpu.sync_copy(rows_v, out_hbm.at[pl.ds(r0, _G)])


def kernel(x, char_lookup):
    idx = x.astype(jnp.int32).reshape(_NROWS, _LANE)
    out = _emb_gather(idx, char_lookup)
    return out.reshape(BATCH, SEQ_LEN, CHAR_DIM)


# speedup vs baseline: 4.1310x; 4.1310x over previous
import functools

import jax
import jax.numpy as jnp
from jax import lax
from jax.experimental import pallas as pl
from jax.experimental.pallas import tpu as pltpu
from jax.experimental.pallas import tpu_sc as plsc

VOCAB = 100000
CHAR_DIM = 64
BATCH = 4096
SEQ_LEN = 200

_N = BATCH * SEQ_LEN
_LANE = 128
_NROWS = _N // _LANE
_NW = 32
_IROWS_W = _NROWS // _NW
_G = 8
_NBLK = _IROWS_W // _G


@functools.partial(
    pl.kernel,
    out_type=jax.ShapeDtypeStruct((_NROWS, _LANE, CHAR_DIM), jnp.float32),
    mesh=plsc.VectorSubcoreMesh(core_axis_name="c", subcore_axis_name="s"),
    scratch_types=[
        pltpu.VMEM((_G, _LANE), jnp.int32),
        pltpu.VMEM((_G, _LANE, CHAR_DIM), jnp.float32),
        pltpu.SemaphoreType.DMA,
    ],
    compiler_params=pltpu.CompilerParams(use_tc_tiling_on_sc=False),
)
def _emb_gather(idx_hbm, tab_hbm, out_hbm, idx_v, rows_v, sem):
    num_cores = 2
    wid = lax.axis_index("s") * num_cores + lax.axis_index("c")
    base = wid * _IROWS_W

    @pl.loop(0, _NBLK)
    def _blk(b):
        r0 = base + b * _G
        pltpu.sync_copy(idx_hbm.at[pl.ds(r0, _G)], idx_v)
        copies = [
            pltpu.async_copy(tab_hbm.at[idx_v.at[j]], rows_v.at[j], sem)
            for j in range(_G)
        ]
        for c in copies:
            c.wait()
        pltpu.sync_copy(rows_v, out_hbm.at[pl.ds(r0, _G)])


def kernel(x, char_lookup):
    idx = x.astype(jnp.int32).reshape(_NROWS, _LANE)
    out = _emb_gather(idx, char_lookup)
    return out.reshape(BATCH, SEQ_LEN, CHAR_DIM)


# speedup vs baseline: 4.2478x; 1.0283x over previous
import functools

import jax
import jax.numpy as jnp
from jax import lax
from jax.experimental import pallas as pl
from jax.experimental.pallas import tpu as pltpu
from jax.experimental.pallas import tpu_sc as plsc

VOCAB = 100000
CHAR_DIM = 64
BATCH = 4096
SEQ_LEN = 200

_N = BATCH * SEQ_LEN
_LANE = 128
_NROWS = _N // _LANE
_NW = 32
_IROWS_W = _NROWS // _NW
_G = 5
_NBLK = _IROWS_W // _G


@functools.partial(
    pl.kernel,
    out_type=jax.ShapeDtypeStruct((_NROWS, _LANE, CHAR_DIM), jnp.float32),
    mesh=plsc.VectorSubcoreMesh(core_axis_name="c", subcore_axis_name="s"),
    scratch_types=[
        pltpu.VMEM((2, _G, _LANE), jnp.int32),
        pltpu.VMEM((2, _G, _LANE, CHAR_DIM), jnp.float32),
        pltpu.SemaphoreType.DMA,
        pltpu.SemaphoreType.DMA,
        pltpu.SemaphoreType.DMA,
    ],
    compiler_params=pltpu.CompilerParams(use_tc_tiling_on_sc=False),
)
def _emb_gather(idx_hbm, tab_hbm, out_hbm, idx_v, rows_v, sem_i, sem_g, sem_o):
    num_cores = 2
    wid = lax.axis_index("s") * num_cores + lax.axis_index("c")
    base = wid * _IROWS_W
    last = base + (_NBLK - 1) * _G

    pltpu.sync_copy(idx_hbm.at[pl.ds(base, _G)], idx_v.at[0])
    pltpu.async_copy(idx_hbm.at[pl.ds(base + _G, _G)], idx_v.at[1], sem_i)

    @pl.loop(0, _NBLK // 2)
    def _pair(p):
        for ph in range(2):
            cur, nxt = ph, 1 - ph
            b = 2 * p + ph
            r0 = base + b * _G
            gathers = [
                pltpu.async_copy(
                    tab_hbm.at[idx_v.at[cur].at[j]], rows_v.at[cur].at[j], sem_g
                )
                for j in range(_G)
            ]
            pltpu.make_async_copy(
                idx_hbm.at[pl.ds(base, _G)], idx_v.at[nxt], sem_i
            ).wait()
            for c in gathers:
                c.wait()
            r2 = jnp.minimum(r0 + 2 * _G, last)
            pltpu.async_copy(idx_hbm.at[pl.ds(r2, _G)], idx_v.at[cur], sem_i)

            @pl.when(b > 0)
            def _():
                pltpu.make_async_copy(
                    rows_v.at[nxt], out_hbm.at[pl.ds(base, _G)], sem_o
                ).wait()

            pltpu.async_copy(rows_v.at[cur], out_hbm.at[pl.ds(r0, _G)], sem_o)

    pltpu.make_async_copy(rows_v.at[1], out_hbm.at[pl.ds(base, _G)], sem_o).wait()
    pltpu.make_async_copy(idx_hbm.at[pl.ds(base, _G)], idx_v.at[0], sem_i).wait()


def kernel(x, char_lookup):
    idx = x.astype(jnp.int32).reshape(_NROWS, _LANE)
    out = _emb_gather(idx, char_lookup)
    return out.reshape(BATCH, SEQ_LEN, CHAR_DIM)
